# baseline (speedup 1.0000x reference)
import dataclasses
import functools

import jax
import jax.numpy as jnp
from jax import lax
from jax.experimental import pallas as pl
from jax.experimental.pallas import tpu as pltpu
from jax.experimental.pallas import tpu_sc as plsc

N = 10000
NP = 10240
D = 128
E = 320000
NW = 32
RB = 2048

_mesh = plsc.VectorSubcoreMesh(core_axis_name="c", subcore_axis_name="s")

_sc_params = pltpu.CompilerParams()
if "needs_layout_passes" in pltpu.CompilerParams.__dataclass_fields__:
    _sc_params = dataclasses.replace(_sc_params, needs_layout_passes=False)
_sc_params = dataclasses.replace(_sc_params, use_tc_tiling_on_sc=False)


QD = D // 4
ET = E // 16
CH = 2000
GW = 80
NWIN = CH // GW
NPT = NP // 16


@functools.partial(
    pl.kernel,
    out_type=[
        jax.ShapeDtypeStruct((4 * NP, QD), jnp.float32),
        jax.ShapeDtypeStruct((NW, NP), jnp.float32),
    ],
    mesh=_mesh,
    compiler_params=_sc_params,
    scratch_types=[
        pltpu.VMEM((NP,), jnp.float32),
        pltpu.VMEM((NP,), jnp.float32),
        pltpu.VMEM((NP,), jnp.float32),
        pltpu.VMEM((16,), jnp.float32),
        pltpu.VMEM((2, CH), jnp.int32),
        pltpu.VMEM((2, CH), jnp.int32),
        pltpu.VMEM((NWIN, GW), jnp.int32),
        pltpu.VMEM((CH,), jnp.int32),
        pltpu.VMEM((CH,), jnp.float32),
        pltpu.VMEM((CH, QD), jnp.float32),
        pltpu.VMEM_SHARED((NP, QD), jnp.float32),
        pltpu.SemaphoreType.DMA,
        pltpu.SemaphoreType.DMA,
        pltpu.SemaphoreType.DMA,
    ],
)
def _sc_edges(src_hbm, dst_hbm, asrc_hbm, adst_hbm, mb_hbm, h4_hbm,
              racc_hbm, dpart_hbm,
              asrc_v, adst_v, den_v, mb_v, src_v, dst_v, dst2_v, idx_v, w_v,
              rows_v, acc_sh, sem, sem2, sem_i):
    cid = lax.axis_index("c")
    sid = lax.axis_index("s")
    wid = sid * 2 + cid

    pltpu.sync_copy(asrc_hbm, asrc_v)
    pltpu.sync_copy(adst_hbm, adst_v)
    pltpu.sync_copy(mb_hbm, mb_v)

    z16 = jnp.zeros((16,), jnp.float32)

    @pl.loop(0, NP, step=16)
    def _(i):
        den_v[pl.ds(i, 16)] = z16

    m16 = mb_v[pl.ds(0, 16)]
    base = sid * ET
    row0 = sid * NPT

    def fire_idx(c0, b):
        pltpu.async_copy(src_hbm.at[pl.ds(base + c0, CH)], src_v.at[b], sem_i)
        pltpu.async_copy(dst_hbm.at[pl.ds(base + c0, CH)], dst_v.at[b], sem_i)

    def drain_idx(c0, b):
        pltpu.make_async_copy(src_hbm.at[pl.ds(base + c0, CH)],
                              src_v.at[b], sem_i).wait()
        pltpu.make_async_copy(dst_hbm.at[pl.ds(base + c0, CH)],
                              dst_v.at[b], sem_i).wait()

    def process(c0, b, q):
        sb = src_v.at[b]
        db = dst_v.at[b]

        @pl.loop(0, NWIN)
        def _(j):
            @pl.loop(0, GW, step=16)
            def _(t):
                i = j * GW + t
                s16 = sb[pl.ds(i, 16)]
                d16 = db[pl.ds(i, 16)]
                a = (plsc.load_gather(asrc_v, [s16])
                     + plsc.load_gather(adst_v, [d16]))
                a = jnp.where(a >= 0.0, a, 0.2 * a)
                w = jnp.exp(a - m16)
                w_v[pl.ds(i, 16)] = w
                plsc.addupdate_scatter(den_v, [d16], w)
                idx_v[pl.ds(i, 16)] = s16 * 4 + q
                dst2_v[j, pl.ds(t, 16)] = d16

            pltpu.async_copy(
                h4_hbm.at[idx_v.at[pl.ds(j * GW, GW)]],
                rows_v.at[pl.ds(j * GW, GW)], sem)

        @pl.loop(0, NWIN)
        def _(j):
            pltpu.make_async_copy(
                h4_hbm.at[idx_v.at[pl.ds(j * GW, GW)]],
                rows_v.at[pl.ds(j * GW, GW)], sem).wait()

            @pl.loop(0, GW)
            def _(t):
                r = j * GW + t
                ws = plsc.load_gather(w_v, [lax.broadcast(r, (16,))])
                for c in range(QD // 16):
                    sl = (r, pl.ds(c * 16, 16))
                    rows_v[sl] = rows_v[sl] * ws

            pltpu.async_copy(rows_v.at[pl.ds(j * GW, GW)],
                             acc_sh.at[dst2_v.at[j]], sem2, add=True)

        @pl.loop(0, NWIN)
        def _(j):
            pltpu.make_async_copy(rows_v.at[pl.ds(j * GW, GW)],
                                  acc_sh.at[dst2_v.at[j]], sem2).wait()

    for p in range(2):
        q = cid * 2 + p

        @pl.loop(0, NPT)
        def _(r):
            for c in range(QD // 16):
                rows_v[r, pl.ds(c * 16, 16)] = z16

        pltpu.sync_copy(rows_v.at[pl.ds(0, NPT)], acc_sh.at[pl.ds(row0, NPT)])
        plsc.subcore_barrier()

        fire_idx(0, 0)

        @pl.loop(0, ET, step=2 * CH)
        def _(c0):
            fire_idx(c0 + CH, 1)
            drain_idx(c0, 0)
            process(c0, 0, q)

            @pl.when(c0 + 2 * CH < ET)
            def _():
                fire_idx(c0 + 2 * CH, 0)

            drain_idx(c0 + CH, 1)
            process(c0 + CH, 1, q)

        plsc.subcore_barrier()
        pltpu.sync_copy(acc_sh.at[pl.ds(row0, NPT)],
                        racc_hbm.at[pl.ds(q * NP + row0, NPT)])
        plsc.subcore_barrier()

    pltpu.sync_copy(den_v, dpart_hbm.at[wid])


def _prep_body(x_ref, w_ref, av_ref, bv_ref,
               h_ref, asrc_ref, adst_ref, mb_ref, selfw_ref):
    h = jnp.dot(x_ref[...], w_ref[...], preferred_element_type=jnp.float32)
    h_ref[...] = h
    s = jnp.sum(h * av_ref[...][None, :], axis=1)
    t = jnp.sum(h * bv_ref[...][None, :], axis=1)
    asrc_ref[...] = s
    adst_ref[...] = t
    u = s + t
    sal = jnp.where(u >= 0.0, u, 0.2 * u)
    mraw = jnp.max(s) + jnp.max(t)
    m = jnp.where(mraw >= 0.0, mraw, 0.2 * mraw)
    mb_ref[...] = jnp.full((16,), m, jnp.float32)
    selfw_ref[...] = jnp.exp(sal - m)


def _prep(x, W, a_src, a_dst):
    return pl.pallas_call(
        _prep_body,
        grid=(1,),
        in_specs=[
            pl.BlockSpec((NP, D), lambda i: (0, 0)),
            pl.BlockSpec((D, D), lambda i: (0, 0)),
            pl.BlockSpec((D,), lambda i: (0,)),
            pl.BlockSpec((D,), lambda i: (0,)),
        ],
        out_specs=[
            pl.BlockSpec((NP, D), lambda i: (0, 0)),
            pl.BlockSpec((NP,), lambda i: (0,)),
            pl.BlockSpec((NP,), lambda i: (0,)),
            pl.BlockSpec((16,), lambda i: (0,)),
            pl.BlockSpec((NP,), lambda i: (0,)),
        ],
        out_shape=[
            jax.ShapeDtypeStruct((NP, D), jnp.float32),
            jax.ShapeDtypeStruct((NP,), jnp.float32),
            jax.ShapeDtypeStruct((NP,), jnp.float32),
            jax.ShapeDtypeStruct((16,), jnp.float32),
            jax.ShapeDtypeStruct((NP,), jnp.float32),
        ],
    )(x, W, a_src, a_dst)


def _finish_body(apply_gelu, r0_ref, r1_ref, r2_ref, r3_ref, dp_ref, sw_ref,
                 h_ref, b_ref, out_ref):
    sw = sw_ref[...]
    den = 0.25 * jnp.sum(dp_ref[...], axis=0) + sw
    acc = jnp.concatenate(
        [r0_ref[...], r1_ref[...], r2_ref[...], r3_ref[...]], axis=1)
    num = acc + sw[:, None] * h_ref[...]
    out = num / den[:, None] + b_ref[...][None, :]
    if apply_gelu:
        out = 0.5 * out * (1.0 + lax.erf(out * (2.0 ** -0.5)))
    out_ref[...] = out


def _finish(racc, dpart, selfw, h, b, apply_gelu):
    nb = NP // RB

    def qspec(q):
        return pl.BlockSpec((RB, QD), lambda i, q=q: (i + q * nb, 0))

    return pl.pallas_call(
        functools.partial(_finish_body, apply_gelu),
        grid=(nb,),
        in_specs=[
            qspec(0), qspec(1), qspec(2), qspec(3),
            pl.BlockSpec((NW, RB), lambda i: (0, i)),
            pl.BlockSpec((RB,), lambda i: (i,)),
            pl.BlockSpec((RB, D), lambda i: (i, 0)),
            pl.BlockSpec((D,), lambda i: (0,)),
        ],
        out_specs=pl.BlockSpec((RB, D), lambda i: (i, 0)),
        out_shape=jax.ShapeDtypeStruct((NP, D), jnp.float32),
    )(racc, racc, racc, racc, dpart, selfw, h, b)


def kernel(embeded_nodes_features, edges_connectivity,
           W0, a_src0, a_dst0, b0, W1, a_src1, a_dst1, b1):
    src = edges_connectivity[0].astype(jnp.int32)
    dst = edges_connectivity[1].astype(jnp.int32)

    def layer(xin, W, a_s, a_d, b, apply_gelu):
        h, asrc, adst, mb, selfw = _prep(xin, W, a_s, a_d)
        h4 = jnp.reshape(h, (4 * NP, QD))
        racc, dpart = _sc_edges(src, dst, asrc, adst, mb, h4)
        return _finish(racc, dpart, selfw, h, b, apply_gelu)

    xp = jnp.zeros((NP, D), jnp.float32).at[:N].set(embeded_nodes_features)
    x1 = layer(xp, W0, a_src0, a_dst0, b0, True)
    return layer(x1, W1, a_src1, a_dst1, b1, False)[:N]


# speedup vs baseline: 1.1826x; 1.1826x over previous
import dataclasses
import functools

import jax
import jax.numpy as jnp
from jax import lax
from jax.experimental import pallas as pl
from jax.experimental.pallas import tpu as pltpu
from jax.experimental.pallas import tpu_sc as plsc

N = 10000
NP = 10240
D = 128
E = 320000
NW = 32
RB = 2048

_mesh = plsc.VectorSubcoreMesh(core_axis_name="c", subcore_axis_name="s")

_sc_params = pltpu.CompilerParams()
if "needs_layout_passes" in pltpu.CompilerParams.__dataclass_fields__:
    _sc_params = dataclasses.replace(_sc_params, needs_layout_passes=False)
_sc_params = dataclasses.replace(_sc_params, use_tc_tiling_on_sc=False)


QD = D // 4
ET = E // 16
CH = 2000
GW = 80
NWIN = CH // GW
NPT = NP // 16


@functools.partial(
    pl.kernel,
    out_type=[
        jax.ShapeDtypeStruct((4 * NP, QD), jnp.float32),
        jax.ShapeDtypeStruct((NW, NP), jnp.float32),
    ],
    mesh=_mesh,
    compiler_params=_sc_params,
    scratch_types=[
        pltpu.VMEM((NP,), jnp.float32),
        pltpu.VMEM((NP,), jnp.float32),
        pltpu.VMEM((NP,), jnp.float32),
        pltpu.VMEM((16,), jnp.float32),
        pltpu.VMEM((2, CH), jnp.int32),
        pltpu.VMEM((2, CH), jnp.int32),
        pltpu.VMEM((NWIN, GW), jnp.int32),
        pltpu.VMEM((CH,), jnp.int32),
        pltpu.VMEM((CH,), jnp.float32),
        pltpu.VMEM((CH, QD), jnp.float32),
        pltpu.VMEM_SHARED((NP, QD), jnp.float32),
        pltpu.SemaphoreType.DMA,
        pltpu.SemaphoreType.DMA,
        pltpu.SemaphoreType.DMA,
    ],
)
def _sc_edges(src_hbm, dst_hbm, asrc_hbm, adst_hbm, mb_hbm, h4_hbm,
              racc_hbm, dpart_hbm,
              asrc_v, adst_v, den_v, mb_v, src_v, dst_v, dst2_v, idx_v, w_v,
              rows_v, acc_sh, sem, sem2, sem_i):
    cid = lax.axis_index("c")
    sid = lax.axis_index("s")
    wid = sid * 2 + cid

    pltpu.sync_copy(asrc_hbm, asrc_v)
    pltpu.sync_copy(adst_hbm, adst_v)
    pltpu.sync_copy(mb_hbm, mb_v)

    z16 = jnp.zeros((16,), jnp.float32)

    @pl.loop(0, NP, step=16)
    def _(i):
        den_v[pl.ds(i, 16)] = z16

    m16 = mb_v[pl.ds(0, 16)]
    base = sid * ET
    row0 = sid * NPT

    def fire_idx(c0, b):
        pltpu.async_copy(src_hbm.at[pl.ds(base + c0, CH)], src_v.at[b], sem_i)
        pltpu.async_copy(dst_hbm.at[pl.ds(base + c0, CH)], dst_v.at[b], sem_i)

    def drain_idx(c0, b):
        pltpu.make_async_copy(src_hbm.at[pl.ds(base + c0, CH)],
                              src_v.at[b], sem_i).wait()
        pltpu.make_async_copy(dst_hbm.at[pl.ds(base + c0, CH)],
                              dst_v.at[b], sem_i).wait()

    def process(c0, b, q):
        sb = src_v.at[b]
        db = dst_v.at[b]

        @pl.loop(0, NWIN)
        def _(j):
            @pl.loop(0, GW, step=16)
            def _(t):
                i = j * GW + t
                s16 = sb[pl.ds(i, 16)]
                d16 = db[pl.ds(i, 16)]
                a = (plsc.load_gather(asrc_v, [s16])
                     + plsc.load_gather(adst_v, [d16]))
                a = jnp.where(a >= 0.0, a, 0.2 * a)
                w = jnp.exp(a - m16)
                w_v[pl.ds(i, 16)] = w
                plsc.addupdate_scatter(den_v, [d16], w)
                idx_v[pl.ds(i, 16)] = s16 * 4 + q
                dst2_v[j, pl.ds(t, 16)] = d16

            pltpu.async_copy(
                h4_hbm.at[idx_v.at[pl.ds(j * GW, GW)]],
                rows_v.at[pl.ds(j * GW, GW)], sem)

        @pl.loop(0, NWIN)
        def _(j):
            pltpu.make_async_copy(
                h4_hbm.at[idx_v.at[pl.ds(j * GW, GW)]],
                rows_v.at[pl.ds(j * GW, GW)], sem).wait()

        @pl.loop(0, CH)
        def _(r):
            ws = plsc.load_gather(w_v, [lax.broadcast(r, (16,))])
            for c in range(QD // 16):
                sl = (r, pl.ds(c * 16, 16))
                rows_v[sl] = rows_v[sl] * ws

        @pl.loop(0, NWIN)
        def _(j):
            pltpu.async_copy(rows_v.at[pl.ds(j * GW, GW)],
                             acc_sh.at[dst2_v.at[j]], sem2, add=True)

        @pl.loop(0, NWIN)
        def _(j):
            pltpu.make_async_copy(rows_v.at[pl.ds(j * GW, GW)],
                                  acc_sh.at[dst2_v.at[j]], sem2).wait()

    for p in range(2):
        q = cid * 2 + p

        @pl.loop(0, NPT)
        def _(r):
            for c in range(QD // 16):
                rows_v[r, pl.ds(c * 16, 16)] = z16

        pltpu.sync_copy(rows_v.at[pl.ds(0, NPT)], acc_sh.at[pl.ds(row0, NPT)])
        plsc.subcore_barrier()

        fire_idx(0, 0)

        @pl.loop(0, ET, step=2 * CH)
        def _(c0):
            fire_idx(c0 + CH, 1)
            drain_idx(c0, 0)
            process(c0, 0, q)

            @pl.when(c0 + 2 * CH < ET)
            def _():
                fire_idx(c0 + 2 * CH, 0)

            drain_idx(c0 + CH, 1)
            process(c0 + CH, 1, q)

        plsc.subcore_barrier()
        pltpu.sync_copy(acc_sh.at[pl.ds(row0, NPT)],
                        racc_hbm.at[pl.ds(q * NP + row0, NPT)])
        plsc.subcore_barrier()

    pltpu.sync_copy(den_v, dpart_hbm.at[wid])


def _prep_body(x_ref, w_ref, av_ref, bv_ref,
               h_ref, asrc_ref, adst_ref, mb_ref, selfw_ref):
    h = jnp.dot(x_ref[...], w_ref[...], preferred_element_type=jnp.float32)
    h_ref[...] = h
    s = jnp.sum(h * av_ref[...][None, :], axis=1)
    t = jnp.sum(h * bv_ref[...][None, :], axis=1)
    asrc_ref[...] = s
    adst_ref[...] = t
    u = s + t
    sal = jnp.where(u >= 0.0, u, 0.2 * u)
    mraw = jnp.max(s) + jnp.max(t)
    m = jnp.where(mraw >= 0.0, mraw, 0.2 * mraw)
    mb_ref[...] = jnp.full((16,), m, jnp.float32)
    selfw_ref[...] = jnp.exp(sal - m)


def _prep(x, W, a_src, a_dst):
    return pl.pallas_call(
        _prep_body,
        grid=(1,),
        in_specs=[
            pl.BlockSpec((NP, D), lambda i: (0, 0)),
            pl.BlockSpec((D, D), lambda i: (0, 0)),
            pl.BlockSpec((D,), lambda i: (0,)),
            pl.BlockSpec((D,), lambda i: (0,)),
        ],
        out_specs=[
            pl.BlockSpec((NP, D), lambda i: (0, 0)),
            pl.BlockSpec((NP,), lambda i: (0,)),
            pl.BlockSpec((NP,), lambda i: (0,)),
            pl.BlockSpec((16,), lambda i: (0,)),
            pl.BlockSpec((NP,), lambda i: (0,)),
        ],
        out_shape=[
            jax.ShapeDtypeStruct((NP, D), jnp.float32),
            jax.ShapeDtypeStruct((NP,), jnp.float32),
            jax.ShapeDtypeStruct((NP,), jnp.float32),
            jax.ShapeDtypeStruct((16,), jnp.float32),
            jax.ShapeDtypeStruct((NP,), jnp.float32),
        ],
    )(x, W, a_src, a_dst)


def _finish_body(apply_gelu, r0_ref, r1_ref, r2_ref, r3_ref, dp_ref, sw_ref,
                 h_ref, b_ref, out_ref):
    sw = sw_ref[...]
    den = 0.25 * jnp.sum(dp_ref[...], axis=0) + sw
    acc = jnp.concatenate(
        [r0_ref[...], r1_ref[...], r2_ref[...], r3_ref[...]], axis=1)
    num = acc + sw[:, None] * h_ref[...]
    out = num / den[:, None] + b_ref[...][None, :]
    if apply_gelu:
        out = 0.5 * out * (1.0 + lax.erf(out * (2.0 ** -0.5)))
    out_ref[...] = out


def _finish(racc, dpart, selfw, h, b, apply_gelu):
    nb = NP // RB

    def qspec(q):
        return pl.BlockSpec((RB, QD), lambda i, q=q: (i + q * nb, 0))

    return pl.pallas_call(
        functools.partial(_finish_body, apply_gelu),
        grid=(nb,),
        in_specs=[
            qspec(0), qspec(1), qspec(2), qspec(3),
            pl.BlockSpec((NW, RB), lambda i: (0, i)),
            pl.BlockSpec((RB,), lambda i: (i,)),
            pl.BlockSpec((RB, D), lambda i: (i, 0)),
            pl.BlockSpec((D,), lambda i: (0,)),
        ],
        out_specs=pl.BlockSpec((RB, D), lambda i: (i, 0)),
        out_shape=jax.ShapeDtypeStruct((NP, D), jnp.float32),
    )(racc, racc, racc, racc, dpart, selfw, h, b)


def kernel(embeded_nodes_features, edges_connectivity,
           W0, a_src0, a_dst0, b0, W1, a_src1, a_dst1, b1):
    src = edges_connectivity[0].astype(jnp.int32)
    dst = edges_connectivity[1].astype(jnp.int32)

    def layer(xin, W, a_s, a_d, b, apply_gelu):
        h, asrc, adst, mb, selfw = _prep(xin, W, a_s, a_d)
        h4 = jnp.reshape(h, (4 * NP, QD))
        racc, dpart = _sc_edges(src, dst, asrc, adst, mb, h4)
        return _finish(racc, dpart, selfw, h, b, apply_gelu)

    xp = jnp.zeros((NP, D), jnp.float32).at[:N].set(embeded_nodes_features)
    x1 = layer(xp, W0, a_src0, a_dst0, b0, True)
    return layer(x1, W1, a_src1, a_dst1, b1, False)[:N]


# speedup vs baseline: 1.1892x; 1.0056x over previous
import dataclasses
import functools

import jax
import jax.numpy as jnp
from jax import lax
from jax.experimental import pallas as pl
from jax.experimental.pallas import tpu as pltpu
from jax.experimental.pallas import tpu_sc as plsc

N = 10000
NP = 10240
D = 128
E = 320000
NW = 32
RB = 2048

_mesh = plsc.VectorSubcoreMesh(core_axis_name="c", subcore_axis_name="s")

_sc_params = pltpu.CompilerParams()
if "needs_layout_passes" in pltpu.CompilerParams.__dataclass_fields__:
    _sc_params = dataclasses.replace(_sc_params, needs_layout_passes=False)
_sc_params = dataclasses.replace(_sc_params, use_tc_tiling_on_sc=False)


QD = D // 4
ET = E // 16
CH = 2000
GW = 80
NWIN = CH // GW
NPT = NP // 16


@functools.partial(
    pl.kernel,
    out_type=[
        jax.ShapeDtypeStruct((4 * NP, QD), jnp.float32),
        jax.ShapeDtypeStruct((NW, NP), jnp.float32),
    ],
    mesh=_mesh,
    compiler_params=_sc_params,
    scratch_types=[
        pltpu.VMEM((NP,), jnp.float32),
        pltpu.VMEM((NP,), jnp.float32),
        pltpu.VMEM((NP,), jnp.float32),
        pltpu.VMEM((16,), jnp.float32),
        pltpu.VMEM((2, CH), jnp.int32),
        pltpu.VMEM((2, CH), jnp.int32),
        pltpu.VMEM((NWIN, GW), jnp.int32),
        pltpu.VMEM((CH,), jnp.int32),
        pltpu.VMEM((CH,), jnp.float32),
        pltpu.VMEM((CH, QD), jnp.float32),
        pltpu.VMEM_SHARED((NP, QD), jnp.float32),
        pltpu.SemaphoreType.DMA,
        pltpu.SemaphoreType.DMA,
        pltpu.SemaphoreType.DMA,
    ],
)
def _sc_edges(src_hbm, dst_hbm, asrc_hbm, adst_hbm, mb_hbm, h4_hbm,
              racc_hbm, dpart_hbm,
              asrc_v, adst_v, den_v, mb_v, src_v, dst_v, dst2_v, idx_v, w_v,
              rows_v, acc_sh, sem, sem2, sem_i):
    cid = lax.axis_index("c")
    sid = lax.axis_index("s")
    wid = sid * 2 + cid

    pltpu.sync_copy(asrc_hbm, asrc_v)
    pltpu.sync_copy(adst_hbm, adst_v)
    pltpu.sync_copy(mb_hbm, mb_v)

    z16 = jnp.zeros((16,), jnp.float32)

    @pl.loop(0, NP, step=16)
    def _(i):
        den_v[pl.ds(i, 16)] = z16

    m16 = mb_v[pl.ds(0, 16)]
    base = sid * ET
    row0 = sid * NPT

    def fire_idx(c0, b):
        pltpu.async_copy(src_hbm.at[pl.ds(base + c0, CH)], src_v.at[b], sem_i)
        pltpu.async_copy(dst_hbm.at[pl.ds(base + c0, CH)], dst_v.at[b], sem_i)

    def drain_idx(c0, b):
        pltpu.make_async_copy(src_hbm.at[pl.ds(base + c0, CH)],
                              src_v.at[b], sem_i).wait()
        pltpu.make_async_copy(dst_hbm.at[pl.ds(base + c0, CH)],
                              dst_v.at[b], sem_i).wait()

    def process(c0, b, q):
        sb = src_v.at[b]
        db = dst_v.at[b]

        @pl.loop(0, NWIN)
        def _(j):
            @pl.loop(0, GW, step=16)
            def _(t):
                i = j * GW + t
                s16 = sb[pl.ds(i, 16)]
                d16 = db[pl.ds(i, 16)]
                a = (plsc.load_gather(asrc_v, [s16])
                     + plsc.load_gather(adst_v, [d16]))
                a = jnp.where(a >= 0.0, a, 0.2 * a)
                w = jnp.exp(a - m16)
                w_v[pl.ds(i, 16)] = w
                plsc.addupdate_scatter(den_v, [d16], w)
                idx_v[pl.ds(i, 16)] = s16 * 4 + q
                dst2_v[j, pl.ds(t, 16)] = d16

            pltpu.async_copy(
                h4_hbm.at[idx_v.at[pl.ds(j * GW, GW)]],
                rows_v.at[pl.ds(j * GW, GW)], sem)

        @pl.loop(0, NWIN)
        def _(j):
            pltpu.make_async_copy(
                h4_hbm.at[idx_v.at[pl.ds(j * GW, GW)]],
                rows_v.at[pl.ds(j * GW, GW)], sem).wait()

        @pl.loop(0, CH)
        def _(r):
            ws = plsc.load_gather(w_v, [lax.broadcast(r, (16,))])
            for c in range(QD // 16):
                sl = (r, pl.ds(c * 16, 16))
                rows_v[sl] = rows_v[sl] * ws

        @pl.loop(0, NWIN)
        def _(j):
            pltpu.async_copy(rows_v.at[pl.ds(j * GW, GW)],
                             acc_sh.at[dst2_v.at[j]], sem2, add=True)

        @pl.loop(0, NWIN)
        def _(j):
            pltpu.make_async_copy(rows_v.at[pl.ds(j * GW, GW)],
                                  acc_sh.at[dst2_v.at[j]], sem2).wait()

    for p in range(2):
        q = cid * 2 + p

        @pl.loop(0, NPT)
        def _(r):
            for c in range(QD // 16):
                rows_v[r, pl.ds(c * 16, 16)] = z16

        pltpu.sync_copy(rows_v.at[pl.ds(0, NPT)], acc_sh.at[pl.ds(row0, NPT)])
        plsc.subcore_barrier()

        fire_idx(0, 0)

        @pl.loop(0, ET, step=2 * CH)
        def _(c0):
            fire_idx(c0 + CH, 1)
            drain_idx(c0, 0)
            process(c0, 0, q)

            @pl.when(c0 + 2 * CH < ET)
            def _():
                fire_idx(c0 + 2 * CH, 0)

            drain_idx(c0 + CH, 1)
            process(c0 + CH, 1, q)

        plsc.subcore_barrier()
        pltpu.sync_copy(acc_sh.at[pl.ds(row0, NPT)],
                        racc_hbm.at[pl.ds(q * NP + row0, NPT)])
        plsc.subcore_barrier()

    pltpu.sync_copy(den_v, dpart_hbm.at[wid])


def _prep_body(x_ref, w_ref, av_ref, bv_ref,
               h_ref, asrc_ref, adst_ref, mb_ref, selfw_ref):
    h = jnp.dot(x_ref[...], w_ref[...], preferred_element_type=jnp.float32)
    h_ref[...] = h
    s = jnp.sum(h * av_ref[...][None, :], axis=1)
    t = jnp.sum(h * bv_ref[...][None, :], axis=1)
    asrc_ref[...] = s
    adst_ref[...] = t
    u = s + t
    sal = jnp.where(u >= 0.0, u, 0.2 * u)
    mraw = jnp.max(s) + jnp.max(t)
    m = jnp.where(mraw >= 0.0, mraw, 0.2 * mraw)
    mb_ref[...] = jnp.full((16,), m, jnp.float32)
    selfw_ref[...] = jnp.exp(sal - m)


def _prep(x, W, a_src, a_dst):
    return pl.pallas_call(
        _prep_body,
        grid=(1,),
        in_specs=[
            pl.BlockSpec((NP, D), lambda i: (0, 0)),
            pl.BlockSpec((D, D), lambda i: (0, 0)),
            pl.BlockSpec((D,), lambda i: (0,)),
            pl.BlockSpec((D,), lambda i: (0,)),
        ],
        out_specs=[
            pl.BlockSpec((NP, D), lambda i: (0, 0)),
            pl.BlockSpec((NP,), lambda i: (0,)),
            pl.BlockSpec((NP,), lambda i: (0,)),
            pl.BlockSpec((16,), lambda i: (0,)),
            pl.BlockSpec((NP,), lambda i: (0,)),
        ],
        out_shape=[
            jax.ShapeDtypeStruct((NP, D), jnp.float32),
            jax.ShapeDtypeStruct((NP,), jnp.float32),
            jax.ShapeDtypeStruct((NP,), jnp.float32),
            jax.ShapeDtypeStruct((16,), jnp.float32),
            jax.ShapeDtypeStruct((NP,), jnp.float32),
        ],
    )(x, W, a_src, a_dst)


def _finish_body(apply_gelu, r0_ref, r1_ref, r2_ref, r3_ref, dp_ref, sw_ref,
                 h_ref, b_ref, out_ref):
    sw = sw_ref[...]
    den = 0.25 * jnp.sum(dp_ref[...], axis=0) + sw
    acc = jnp.concatenate(
        [r0_ref[...], r1_ref[...], r2_ref[...], r3_ref[...]], axis=1)
    num = acc + sw[:, None] * h_ref[...]
    out = num / den[:, None] + b_ref[...][None, :]
    if apply_gelu:
        out = 0.5 * out * (1.0 + lax.erf(out * (2.0 ** -0.5)))
    out_ref[...] = out


def _finish(racc, dpart, selfw, h, b, apply_gelu):
    nb = NP // RB

    def qspec(q):
        return pl.BlockSpec((RB, QD), lambda i, q=q: (i + q * nb, 0))

    return pl.pallas_call(
        functools.partial(_finish_body, apply_gelu),
        grid=(nb,),
        in_specs=[
            qspec(0), qspec(1), qspec(2), qspec(3),
            pl.BlockSpec((NW, RB), lambda i: (0, i)),
            pl.BlockSpec((RB,), lambda i: (i,)),
            pl.BlockSpec((RB, D), lambda i: (i, 0)),
            pl.BlockSpec((D,), lambda i: (0,)),
        ],
        out_specs=pl.BlockSpec((RB, D), lambda i: (i, 0)),
        out_shape=jax.ShapeDtypeStruct((NP, D), jnp.float32),
    )(racc, racc, racc, racc, dpart, selfw, h, b)


def _mid_body(r0_ref, r1_ref, r2_ref, r3_ref, dp_ref, sw_ref, h_ref, b_ref,
              w2_ref, av2_ref, bv2_ref,
              h2_ref, asrc2_ref, adst2_ref, mb2_ref, selfw2_ref):
    sw = sw_ref[...]
    den = 0.25 * jnp.sum(dp_ref[...], axis=0) + sw
    acc = jnp.concatenate(
        [r0_ref[...], r1_ref[...], r2_ref[...], r3_ref[...]], axis=1)
    num = acc + sw[:, None] * h_ref[...]
    out = num / den[:, None] + b_ref[...][None, :]
    x1 = 0.5 * out * (1.0 + lax.erf(out * (2.0 ** -0.5)))
    h2 = jnp.dot(x1, w2_ref[...], preferred_element_type=jnp.float32)
    h2_ref[...] = h2
    s = jnp.sum(h2 * av2_ref[...][None, :], axis=1)
    t = jnp.sum(h2 * bv2_ref[...][None, :], axis=1)
    asrc2_ref[...] = s
    adst2_ref[...] = t
    u = s + t
    sal = jnp.where(u >= 0.0, u, 0.2 * u)
    mraw = jnp.max(s) + jnp.max(t)
    m = jnp.where(mraw >= 0.0, mraw, 0.2 * mraw)
    mb2_ref[...] = jnp.full((16,), m, jnp.float32)
    selfw2_ref[...] = jnp.exp(sal - m)


def _mid(racc, dpart, selfw, h, b, W2, a_src2, a_dst2):
    def qspec(q):
        return pl.BlockSpec((NP, QD), lambda i, q=q: (q, 0))

    return pl.pallas_call(
        _mid_body,
        grid=(1,),
        in_specs=[
            qspec(0), qspec(1), qspec(2), qspec(3),
            pl.BlockSpec((NW, NP), lambda i: (0, 0)),
            pl.BlockSpec((NP,), lambda i: (0,)),
            pl.BlockSpec((NP, D), lambda i: (0, 0)),
            pl.BlockSpec((D,), lambda i: (0,)),
            pl.BlockSpec((D, D), lambda i: (0, 0)),
            pl.BlockSpec((D,), lambda i: (0,)),
            pl.BlockSpec((D,), lambda i: (0,)),
        ],
        out_specs=[
            pl.BlockSpec((NP, D), lambda i: (0, 0)),
            pl.BlockSpec((NP,), lambda i: (0,)),
            pl.BlockSpec((NP,), lambda i: (0,)),
            pl.BlockSpec((16,), lambda i: (0,)),
            pl.BlockSpec((NP,), lambda i: (0,)),
        ],
        out_shape=[
            jax.ShapeDtypeStruct((NP, D), jnp.float32),
            jax.ShapeDtypeStruct((NP,), jnp.float32),
            jax.ShapeDtypeStruct((NP,), jnp.float32),
            jax.ShapeDtypeStruct((16,), jnp.float32),
            jax.ShapeDtypeStruct((NP,), jnp.float32),
        ],
    )(racc, racc, racc, racc, dpart, selfw, h, b, W2, a_src2, a_dst2)


def kernel(embeded_nodes_features, edges_connectivity,
           W0, a_src0, a_dst0, b0, W1, a_src1, a_dst1, b1):
    src = edges_connectivity[0].astype(jnp.int32)
    dst = edges_connectivity[1].astype(jnp.int32)

    xp = jnp.zeros((NP, D), jnp.float32).at[:N].set(embeded_nodes_features)

    h1, asrc1, adst1, mb1, selfw1 = _prep(xp, W0, a_src0, a_dst0)
    h1q = jnp.reshape(h1, (4 * NP, QD))
    racc1, dpart1 = _sc_edges(src, dst, asrc1, adst1, mb1, h1q)

    h2, asrc2, adst2, mb2, selfw2 = _mid(
        racc1, dpart1, selfw1, h1, b0, W1, a_src1, a_dst1)
    h2q = jnp.reshape(h2, (4 * NP, QD))
    racc2, dpart2 = _sc_edges(src, dst, asrc2, adst2, mb2, h2q)

    return _finish(racc2, dpart2, selfw2, h2, b1, False)[:N]
